# baseline (speedup 1.0000x reference)
import functools

import jax
import jax.numpy as jnp
from jax import lax
from jax.experimental import pallas as pl
from jax.experimental.pallas import tpu as pltpu
from jax.experimental.pallas import tpu_sc as plsc

NT = 2000
NTP = 2048
D_IN = 2048
H = 128
E = 80000
R = 4
SRC_BLK = [0, 0, 2, 1]

NC, NS = 2, 16
NROW = 63
ACC = NROW * NTP
EC = 512
NRING = 3
E_PAD = 82944
NCHUNK = E_PAD // EC
TOT_CHUNK = R * NCHUNK


def _sc_build_a(pair_ref, a_ref, *scratch):
    bufs = scratch[:NRING]
    acc = scratch[NRING]
    sems = scratch[NRING + 1:]
    c = lax.axis_index("c")
    s = lax.axis_index("s")
    w = c * NS + s
    row0 = w * NROW
    lo = row0 << 11
    hi = (row0 + NROW) << 11
    zeros16 = jnp.zeros((16,), jnp.float32)
    ones16 = jnp.ones((16,), jnp.float32)

    for k in range(NRING):
        pltpu.async_copy(pair_ref.at[pl.ds(k * EC, EC)], bufs[k], sems[k])

    span = jnp.uint32(NROW * NTP)
    for r in range(R):
        @pl.loop(0, ACC // 16, unroll=8)
        def _zero(i):
            acc[pl.ds(i * 16, 16)] = zeros16

        @pl.loop(0, NCHUNK // NRING)
        def _chunks(g):
            for k in range(NRING):
                gc = r * NCHUNK + g * NRING + k
                pltpu.make_async_copy(
                    pair_ref.at[pl.ds(0, EC)], bufs[k], sems[k]).wait()
                for q in range(EC // 16):
                    pv = bufs[k][pl.ds(q * 16, 16)]
                    t = pv - lo
                    m = plsc.bitcast(t, jnp.uint32) < span
                    f = jnp.where(m, t, 0)
                    plsc.addupdate_scatter(acc, [f], ones16, mask=m)

                @pl.when(gc + NRING < TOT_CHUNK)
                def _refill():
                    off = (gc + NRING) * EC
                    pltpu.async_copy(pair_ref.at[pl.ds(off, EC)],
                                     bufs[k], sems[k])

        pltpu.sync_copy(acc, a_ref.at[pl.ds(r * (NTP * NTP) + row0 * NTP, ACC)])


def _build_a(e0, e1, e2, e3):
    pairs = []
    pad_pair = (2 * NT) << 11
    for e in (e0, e1, e2, e3):
        p = e[1] * NTP + e[0]
        pairs.append(jnp.pad(p, (0, E_PAD - E), constant_values=pad_pair))
    pair_flat = jnp.concatenate(pairs)
    mesh = plsc.VectorSubcoreMesh(
        core_axis_name="c", subcore_axis_name="s", num_cores=NC, num_subcores=NS
    )
    scatter = pl.kernel(
        _sc_build_a,
        out_type=jax.ShapeDtypeStruct((R * NTP * NTP,), jnp.float32),
        mesh=mesh,
        compiler_params=pltpu.CompilerParams(needs_layout_passes=False),
        scratch_types=(
            [pltpu.VMEM((EC,), jnp.int32) for _ in range(NRING)]
            + [pltpu.VMEM((ACC,), jnp.float32)]
            + [pltpu.SemaphoreType.DMA for _ in range(NRING)]
        ),
    )
    return scatter(pair_flat)


def _proj_body(x0_ref, x1_ref, x2_ref, x3_ref, x4_ref, p_ref, o_ref):
    t = pl.program_id(0)
    pb = p_ref[0].astype(jnp.bfloat16)
    for i, xr in enumerate((x0_ref, x1_ref, x2_ref, x3_ref, x4_ref)):
        @pl.when(t == i)
        def _do(xr=xr):
            o_ref[0] = jnp.dot(
                xr[...].astype(jnp.bfloat16), pb,
                preferred_element_type=jnp.float32).astype(jnp.bfloat16)


def _proj_all(x0, x1, x2, x3, x4, pstack):
    def xspec(i):
        return pl.BlockSpec((400, D_IN),
                            lambda t, j, i=i: (jnp.where(t == i, j, 0), 0))

    return pl.pallas_call(
        _proj_body,
        grid=(5, 5),
        in_specs=[xspec(0), xspec(1), xspec(2), xspec(3), xspec(4),
                  pl.BlockSpec((1, D_IN, H), lambda t, j: (t, 0, 0))],
        out_specs=pl.BlockSpec((1, 400, H), lambda t, j: (t, j, 0)),
        out_shape=jax.ShapeDtypeStruct((5, NT, H), jnp.bfloat16),
    )(x0, x1, x2, x3, x4, pstack)


def _msg_body(a_ref, h_ref, o_ref):
    a = a_ref[...].reshape(256, NTP)
    hb = h_ref[0]
    hp = jnp.concatenate(
        [hb, jnp.zeros((NTP - NT, H), jnp.bfloat16)], axis=0)
    cnt = jnp.sum(a, axis=1)
    m = jnp.dot(a.astype(jnp.bfloat16), hp,
                preferred_element_type=jnp.float32)
    o_ref[0] = m * (1.0 / jnp.maximum(cnt, 1.0))[:, None]


def _msg(a, hb):
    def h_idx(r, mblk):
        sb = jnp.where(r == 2, 2, jnp.where(r == 3, 1, 0))
        return (sb, 0, 0)

    return pl.pallas_call(
        _msg_body,
        grid=(R, NTP // 256),
        in_specs=[
            pl.BlockSpec((256 * NTP,), lambda r, mblk: (r * 8 + mblk,)),
            pl.BlockSpec((1, NT, H), h_idx),
        ],
        out_specs=pl.BlockSpec((1, 256, H), lambda r, mblk: (r, mblk, 0)),
        out_shape=jax.ShapeDtypeStruct((R, NTP, H), jnp.float32),
    )(a, hb)


def _comb_body(h_ref, msg_ref, comp_ref, bases_ref, root_ref, bias_ref,
               g_ref, b_ref, o_ref, ob_ref):
    blk = pl.program_id(0)
    r = blk - 1
    hb = h_ref[0]
    out = jnp.dot(hb, root_ref[...].astype(jnp.bfloat16),
                  preferred_element_type=jnp.float32)
    out = out + bias_ref[...][None, :]
    comp = comp_ref[...]
    sel = lax.broadcasted_iota(jnp.int32, (R, R), 0) == r
    cr = jnp.sum(jnp.where(sel, comp, 0.0), axis=0)
    bs = bases_ref[...]
    w = jnp.sum(bs * cr[:, None, None], axis=0)
    m = msg_ref[0][:NT]
    out = out + jnp.dot(m.astype(jnp.bfloat16), w.astype(jnp.bfloat16),
                        preferred_element_type=jnp.float32)
    mu = jnp.mean(out, axis=1, keepdims=True)
    v = jnp.mean((out - mu) ** 2, axis=1, keepdims=True)
    y = (out - mu) / jnp.sqrt(v + 1e-5) * g_ref[...][None, :] + b_ref[...][None, :]
    y = jnp.maximum(y, 0.0)
    o_ref[...] = y
    ob_ref[0] = y.astype(jnp.bfloat16)


def _combine(hb, msg, comp, bases, root, bias, g, b):
    return pl.pallas_call(
        _comb_body,
        grid=(5,),
        in_specs=[
            pl.BlockSpec((1, NT, H), lambda blk: (blk, 0, 0)),
            pl.BlockSpec((1, NTP, H), lambda blk: (jnp.maximum(blk - 1, 0), 0, 0)),
            pl.BlockSpec((R, R), lambda blk: (0, 0)),
            pl.BlockSpec((R, H, H), lambda blk: (0, 0, 0)),
            pl.BlockSpec((H, H), lambda blk: (0, 0)),
            pl.BlockSpec((H,), lambda blk: (0,)),
            pl.BlockSpec((H,), lambda blk: (0,)),
            pl.BlockSpec((H,), lambda blk: (0,)),
        ],
        out_specs=[
            pl.BlockSpec((NT, H), lambda blk: (blk, 0)),
            pl.BlockSpec((1, NT, H), lambda blk: (blk, 0, 0)),
        ],
        out_shape=[
            jax.ShapeDtypeStruct((5 * NT, H), jnp.float32),
            jax.ShapeDtypeStruct((5, NT, H), jnp.bfloat16),
        ],
    )(hb, msg, comp, bases, root, bias, g, b)


def kernel(x0, x1, x2, x3, x4, P0, P1, P2, P3, P4, e0, e1, e2, e3,
           comp0, bases0, root0, bias0, gam0, bet0,
           comp1, bases1, root1, bias1, gam1, bet1):
    a = _build_a(e0, e1, e2, e3)
    hb = _proj_all(x0, x1, x2, x3, x4, jnp.stack([P0, P1, P2, P3, P4]))
    y = None
    for comp, bases, root, bias, g, b in [
        (comp0, bases0, root0, bias0, gam0, bet0),
        (comp1, bases1, root1, bias1, gam1, bet1),
    ]:
        msg = _msg(a, hb)
        y, hb = _combine(hb, msg, comp, bases, root, bias, g, b)
    return y


# speedup vs baseline: 1.0433x; 1.0433x over previous
import functools

import jax
import jax.numpy as jnp
from jax import lax
from jax.experimental import pallas as pl
from jax.experimental.pallas import tpu as pltpu
from jax.experimental.pallas import tpu_sc as plsc

NT = 2000
NTP = 2048
D_IN = 2048
H = 128
E = 80000
R = 4
SRC_BLK = [0, 0, 2, 1]

NC, NS = 2, 16
NROW = 63
ACC = NROW * NTP
EC = 256
NRING = 6
E_PAD = 82944
NCHUNK = E_PAD // EC
TOT_CHUNK = R * NCHUNK


def _sc_build_a(pair_ref, a_ref, *scratch):
    bufs = scratch[:NRING]
    acc = scratch[NRING]
    sems = scratch[NRING + 1:]
    c = lax.axis_index("c")
    s = lax.axis_index("s")
    w = c * NS + s
    row0 = w * NROW
    lo = row0 << 11
    hi = (row0 + NROW) << 11
    zeros16 = jnp.zeros((16,), jnp.float32)
    ones16 = jnp.ones((16,), jnp.float32)

    for k in range(NRING):
        pltpu.async_copy(pair_ref.at[pl.ds(k * EC, EC)], bufs[k], sems[k])

    span = jnp.uint32(NROW * NTP)
    for r in range(R):
        @pl.loop(0, ACC // 16, unroll=8)
        def _zero(i):
            acc[pl.ds(i * 16, 16)] = zeros16

        @pl.loop(0, NCHUNK // NRING)
        def _chunks(g):
            for k in range(NRING):
                gc = r * NCHUNK + g * NRING + k
                pltpu.make_async_copy(
                    pair_ref.at[pl.ds(0, EC)], bufs[k], sems[k]).wait()
                for q in range(EC // 16):
                    pv = bufs[k][pl.ds(q * 16, 16)]
                    t = pv - lo
                    m = plsc.bitcast(t, jnp.uint32) < span
                    f = jnp.where(m, t, 0)
                    plsc.addupdate_scatter(acc, [f], ones16, mask=m)

                @pl.when(gc + NRING < TOT_CHUNK)
                def _refill():
                    off = (gc + NRING) * EC
                    pltpu.async_copy(pair_ref.at[pl.ds(off, EC)],
                                     bufs[k], sems[k])

        pltpu.sync_copy(acc, a_ref.at[pl.ds(r * (NTP * NTP) + row0 * NTP, ACC)])


def _build_a(e0, e1, e2, e3):
    pairs = []
    pad_pair = (2 * NT) << 11
    for e in (e0, e1, e2, e3):
        p = e[1] * NTP + e[0]
        pairs.append(jnp.pad(p, (0, E_PAD - E), constant_values=pad_pair))
    pair_flat = jnp.concatenate(pairs)
    mesh = plsc.VectorSubcoreMesh(
        core_axis_name="c", subcore_axis_name="s", num_cores=NC, num_subcores=NS
    )
    scatter = pl.kernel(
        _sc_build_a,
        out_type=jax.ShapeDtypeStruct((R * NTP * NTP,), jnp.float32),
        mesh=mesh,
        compiler_params=pltpu.CompilerParams(needs_layout_passes=False),
        scratch_types=(
            [pltpu.VMEM((EC,), jnp.int32) for _ in range(NRING)]
            + [pltpu.VMEM((ACC,), jnp.float32)]
            + [pltpu.SemaphoreType.DMA for _ in range(NRING)]
        ),
    )
    return scatter(pair_flat)


def _proj_body(x0_ref, x1_ref, x2_ref, x3_ref, x4_ref, p_ref, o_ref):
    t = pl.program_id(0)
    pb = p_ref[0].astype(jnp.bfloat16)
    for i, xr in enumerate((x0_ref, x1_ref, x2_ref, x3_ref, x4_ref)):
        @pl.when(t == i)
        def _do(xr=xr):
            o_ref[0] = jnp.dot(
                xr[...].astype(jnp.bfloat16), pb,
                preferred_element_type=jnp.float32).astype(jnp.bfloat16)


def _proj_all(x0, x1, x2, x3, x4, pstack):
    def xspec(i):
        return pl.BlockSpec((400, D_IN),
                            lambda t, j, i=i: (jnp.where(t == i, j, 0), 0))

    return pl.pallas_call(
        _proj_body,
        grid=(5, 5),
        in_specs=[xspec(0), xspec(1), xspec(2), xspec(3), xspec(4),
                  pl.BlockSpec((1, D_IN, H), lambda t, j: (t, 0, 0))],
        out_specs=pl.BlockSpec((1, 400, H), lambda t, j: (t, j, 0)),
        out_shape=jax.ShapeDtypeStruct((5, NT, H), jnp.bfloat16),
    )(x0, x1, x2, x3, x4, pstack)


def _h_idx(r, mblk):
    sb = jnp.where(r == 2, 2, jnp.where(r == 3, 1, 0))
    return (sb, 0, 0)


def _msg1_body(a_ref, h_ref, o_ref, ab_ref, inv_ref):
    a = a_ref[...].reshape(256, NTP)
    hb = h_ref[0]
    hp = jnp.concatenate(
        [hb, jnp.zeros((NTP - NT, H), jnp.bfloat16)], axis=0)
    cnt = jnp.sum(a, axis=1)
    inv = 1.0 / jnp.maximum(cnt, 1.0)
    ab = a.astype(jnp.bfloat16)
    m = jnp.dot(ab, hp, preferred_element_type=jnp.float32)
    o_ref[0] = m * inv[:, None]
    ab_ref[0] = ab
    inv_ref[0, 0] = inv


def _msg1(a, hb):
    return pl.pallas_call(
        _msg1_body,
        grid=(R, NTP // 256),
        in_specs=[
            pl.BlockSpec((256 * NTP,), lambda r, mblk: (r * 8 + mblk,)),
            pl.BlockSpec((1, NT, H), _h_idx),
        ],
        out_specs=[
            pl.BlockSpec((1, 256, H), lambda r, mblk: (r, mblk, 0)),
            pl.BlockSpec((1, 256, NTP), lambda r, mblk: (r, mblk, 0)),
            pl.BlockSpec((1, 1, 256), lambda r, mblk: (r * 8 + mblk, 0, 0)),
        ],
        out_shape=[
            jax.ShapeDtypeStruct((R, NTP, H), jnp.float32),
            jax.ShapeDtypeStruct((R, NTP, NTP), jnp.bfloat16),
            jax.ShapeDtypeStruct((R * 8, 1, 256), jnp.float32),
        ],
    )(a, hb)


def _msg2_body(ab_ref, inv_ref, h_ref, o_ref):
    hb = h_ref[0]
    hp = jnp.concatenate(
        [hb, jnp.zeros((NTP - NT, H), jnp.bfloat16)], axis=0)
    m = jnp.dot(ab_ref[0], hp, preferred_element_type=jnp.float32)
    o_ref[0] = m * inv_ref[0, 0][:, None]


def _msg2(ab, inv, hb):
    return pl.pallas_call(
        _msg2_body,
        grid=(R, NTP // 256),
        in_specs=[
            pl.BlockSpec((1, 256, NTP), lambda r, mblk: (r, mblk, 0)),
            pl.BlockSpec((1, 1, 256), lambda r, mblk: (r * 8 + mblk, 0, 0)),
            pl.BlockSpec((1, NT, H), _h_idx),
        ],
        out_specs=pl.BlockSpec((1, 256, H), lambda r, mblk: (r, mblk, 0)),
        out_shape=jax.ShapeDtypeStruct((R, NTP, H), jnp.float32),
    )(ab, inv, hb)


def _comb_body(h_ref, msg_ref, comp_ref, bases_ref, root_ref, bias_ref,
               g_ref, b_ref, o_ref, ob_ref):
    blk = pl.program_id(0)
    r = blk - 1
    hb = h_ref[0]
    out = jnp.dot(hb, root_ref[...].astype(jnp.bfloat16),
                  preferred_element_type=jnp.float32)
    out = out + bias_ref[...][None, :]
    comp = comp_ref[...]
    sel = lax.broadcasted_iota(jnp.int32, (R, R), 0) == r
    cr = jnp.sum(jnp.where(sel, comp, 0.0), axis=0)
    bs = bases_ref[...]
    w = jnp.sum(bs * cr[:, None, None], axis=0)
    m = msg_ref[0][:NT]
    out = out + jnp.dot(m.astype(jnp.bfloat16), w.astype(jnp.bfloat16),
                        preferred_element_type=jnp.float32)
    mu = jnp.mean(out, axis=1, keepdims=True)
    v = jnp.mean((out - mu) ** 2, axis=1, keepdims=True)
    y = (out - mu) / jnp.sqrt(v + 1e-5) * g_ref[...][None, :] + b_ref[...][None, :]
    y = jnp.maximum(y, 0.0)
    o_ref[...] = y
    ob_ref[0] = y.astype(jnp.bfloat16)


def _combine(hb, msg, comp, bases, root, bias, g, b):
    return pl.pallas_call(
        _comb_body,
        grid=(5,),
        in_specs=[
            pl.BlockSpec((1, NT, H), lambda blk: (blk, 0, 0)),
            pl.BlockSpec((1, NTP, H), lambda blk: (jnp.maximum(blk - 1, 0), 0, 0)),
            pl.BlockSpec((R, R), lambda blk: (0, 0)),
            pl.BlockSpec((R, H, H), lambda blk: (0, 0, 0)),
            pl.BlockSpec((H, H), lambda blk: (0, 0)),
            pl.BlockSpec((H,), lambda blk: (0,)),
            pl.BlockSpec((H,), lambda blk: (0,)),
            pl.BlockSpec((H,), lambda blk: (0,)),
        ],
        out_specs=[
            pl.BlockSpec((NT, H), lambda blk: (blk, 0)),
            pl.BlockSpec((1, NT, H), lambda blk: (blk, 0, 0)),
        ],
        out_shape=[
            jax.ShapeDtypeStruct((5 * NT, H), jnp.float32),
            jax.ShapeDtypeStruct((5, NT, H), jnp.bfloat16),
        ],
    )(hb, msg, comp, bases, root, bias, g, b)


def kernel(x0, x1, x2, x3, x4, P0, P1, P2, P3, P4, e0, e1, e2, e3,
           comp0, bases0, root0, bias0, gam0, bet0,
           comp1, bases1, root1, bias1, gam1, bet1):
    a = _build_a(e0, e1, e2, e3)
    hb = _proj_all(x0, x1, x2, x3, x4, jnp.stack([P0, P1, P2, P3, P4]))
    msg, ab, inv = _msg1(a, hb)
    _, hb = _combine(hb, msg, comp0, bases0, root0, bias0, gam0, bet0)
    msg = _msg2(ab, inv, hb)
    y, _ = _combine(hb, msg, comp1, bases1, root1, bias1, gam1, bet1)
    return y
